# Initial kernel scaffold; baseline (speedup 1.0000x reference)
import jax
import jax.numpy as jnp
from jax.experimental import pallas as pl


def kernel(x, edge_index, edge_weight, W1, b1, W2, b2):
    raise NotImplementedError("write your pallas kernel here")


# speedup vs baseline: 10.1762x; 10.1762x over previous
import functools

import jax
import jax.numpy as jnp
from jax import lax
from jax.experimental import pallas as pl
from jax.experimental.pallas import tpu as pltpu
from jax.experimental.pallas import tpu_sc as plsc

N = 10000
NP = 10240
D = 128
H = 128
E = 320000
LANES = 128
NW = 32
CH = -(-E // (NW * LANES))
EP = NW * CH * LANES
STRIPE = NP // 16
BR = 640

_mesh = plsc.VectorSubcoreMesh(core_axis_name="c", subcore_axis_name="s")


@functools.partial(
    pl.kernel,
    out_type=jax.ShapeDtypeStruct((2, NP), jnp.float32),
    mesh=_mesh,
    scratch_types=[
        pltpu.VMEM((CH, LANES), jnp.int32),
        pltpu.VMEM((CH, LANES), jnp.float32),
        pltpu.VMEM_SHARED((NP,), jnp.float32),
    ],
)
def _deg_kernel(colp, ewp, zeros_n, out, col_v, ew_v, deg):
    c = lax.axis_index("c")
    s = lax.axis_index("s")
    wid = s * 2 + c
    rows = pl.ds(s * STRIPE, STRIPE)
    pltpu.sync_copy(zeros_n.at[rows], deg.at[rows])
    pltpu.sync_copy(colp.at[wid], col_v)
    pltpu.sync_copy(ewp.at[wid], ew_v)
    plsc.subcore_barrier()

    def body(j, carry):
        pltpu.sync_copy(ew_v.at[j], deg.at[col_v.at[j]], add=True)
        return carry

    lax.fori_loop(0, CH, body, 0)
    plsc.subcore_barrier()
    pltpu.sync_copy(deg.at[rows], out.at[c, rows])


@functools.partial(
    pl.kernel,
    out_type=jax.ShapeDtypeStruct((2, NP, H), jnp.float32),
    mesh=_mesh,
    scratch_types=[
        pltpu.VMEM((CH, LANES), jnp.int32),
        pltpu.VMEM((CH, LANES), jnp.int32),
        pltpu.VMEM((CH, LANES), jnp.float32),
        pltpu.VMEM((LANES, H), jnp.float32),
        pltpu.VMEM_SHARED((NP, H), jnp.float32),
        pltpu.SemaphoreType.DMA,
    ],
)
def _edge_kernel(hp, rowp, colp, ewp, zeros_nh, out, row_v, col_v, ew_v,
                 rows_v, acc, sem):
    c = lax.axis_index("c")
    s = lax.axis_index("s")
    wid = s * 2 + c
    rows = pl.ds(s * STRIPE, STRIPE)

    @pl.when(c == 0)
    def _():
        pltpu.sync_copy(hp.at[rows], acc.at[rows])

    @pl.when(c == 1)
    def _():
        pltpu.sync_copy(zeros_nh.at[rows], acc.at[rows])

    pltpu.sync_copy(rowp.at[wid], row_v)
    pltpu.sync_copy(colp.at[wid], col_v)
    pltpu.sync_copy(ewp.at[wid], ew_v)
    plsc.subcore_barrier()

    def chunk(j, carry):
        pltpu.async_copy(hp.at[row_v.at[j]], rows_v, sem).wait()

        def scale16(g, carry2):
            wv = ew_v[j, pl.ds(g * 16, 16)]
            base = g * 16
            for l in range(16):
                w = wv[l]
                i = base + l
                for k in range(H // 16):
                    sl = pl.ds(k * 16, 16)
                    rows_v[i, sl] = rows_v[i, sl] * w
            return carry2

        lax.fori_loop(0, LANES // 16, scale16, 0)
        pltpu.sync_copy(rows_v, acc.at[col_v.at[j]], add=True)
        return carry

    lax.fori_loop(0, CH, chunk, 0)
    plsc.subcore_barrier()
    pltpu.sync_copy(acc.at[rows], out.at[c, rows])


def _hp1_body(x_ref, w_ref, d0_ref, d1_ref, dis_ref, hp_ref):
    dis = lax.rsqrt(1.0 + d0_ref[...] + d1_ref[...])
    dis_ref[...] = dis
    hp_ref[...] = dis * jnp.dot(x_ref[...], w_ref[...],
                                preferred_element_type=jnp.float32)


def _tc_hp1(xp, W1, d0, d1):
    return pl.pallas_call(
        _hp1_body,
        grid=(NP // BR,),
        in_specs=[
            pl.BlockSpec((BR, D), lambda i: (i, 0)),
            pl.BlockSpec((D, H), lambda i: (0, 0)),
            pl.BlockSpec((BR, 1), lambda i: (i, 0)),
            pl.BlockSpec((BR, 1), lambda i: (i, 0)),
        ],
        out_specs=[
            pl.BlockSpec((BR, 1), lambda i: (i, 0)),
            pl.BlockSpec((BR, H), lambda i: (i, 0)),
        ],
        out_shape=[
            jax.ShapeDtypeStruct((NP, 1), jnp.float32),
            jax.ShapeDtypeStruct((NP, H), jnp.float32),
        ],
    )(xp, W1, d0, d1)


def _mid_body(s0_ref, s1_ref, dis_ref, b_ref, w_ref, hp2_ref):
    z = jnp.maximum(dis_ref[...] * (s0_ref[...] + s1_ref[...]) + b_ref[...],
                    0.0)
    hp2_ref[...] = dis_ref[...] * jnp.dot(z, w_ref[...],
                                          preferred_element_type=jnp.float32)


def _tc_mid(s0, s1, dis, b1, W2):
    return pl.pallas_call(
        _mid_body,
        grid=(NP // BR,),
        in_specs=[
            pl.BlockSpec((BR, H), lambda i: (i, 0)),
            pl.BlockSpec((BR, H), lambda i: (i, 0)),
            pl.BlockSpec((BR, 1), lambda i: (i, 0)),
            pl.BlockSpec((1, H), lambda i: (0, 0)),
            pl.BlockSpec((H, H), lambda i: (0, 0)),
        ],
        out_specs=pl.BlockSpec((BR, H), lambda i: (i, 0)),
        out_shape=jax.ShapeDtypeStruct((NP, H), jnp.float32),
    )(s0, s1, dis, b1, W2)


def _fin_body(s0_ref, s1_ref, dis_ref, b_ref, out_ref):
    out_ref[...] = jnp.maximum(
        dis_ref[...] * (s0_ref[...] + s1_ref[...]) + b_ref[...], 0.0)


def _tc_fin(s0, s1, dis, b2):
    return pl.pallas_call(
        _fin_body,
        grid=(NP // BR,),
        in_specs=[
            pl.BlockSpec((BR, H), lambda i: (i, 0)),
            pl.BlockSpec((BR, H), lambda i: (i, 0)),
            pl.BlockSpec((BR, 1), lambda i: (i, 0)),
            pl.BlockSpec((1, H), lambda i: (0, 0)),
        ],
        out_specs=pl.BlockSpec((BR, H), lambda i: (i, 0)),
        out_shape=jax.ShapeDtypeStruct((NP, H), jnp.float32),
    )(s0, s1, dis, b2)


def kernel(x, edge_index, edge_weight, W1, b1, W2, b2):
    row = edge_index[0]
    col = edge_index[1]
    pe = EP - E
    rowp = jnp.pad(row, (0, pe)).reshape(NW, CH, LANES)
    colp = jnp.pad(col, (0, pe)).reshape(NW, CH, LANES)
    ewp = jnp.pad(edge_weight, (0, pe)).reshape(NW, CH, LANES)
    xp = jnp.pad(x, ((0, NP - N), (0, 0)))
    zn = jnp.zeros((NP,), jnp.float32)
    znh = jnp.zeros((NP, H), jnp.float32)

    degs = _deg_kernel(colp, ewp, zn)
    d0 = degs[0].reshape(NP, 1)
    d1 = degs[1].reshape(NP, 1)
    dis, hp1 = _tc_hp1(xp, W1, d0, d1)

    S = _edge_kernel(hp1, rowp, colp, ewp, znh)
    hp2 = _tc_mid(S[0], S[1], dis, b1.reshape(1, H), W2)

    S2 = _edge_kernel(hp2, rowp, colp, ewp, znh)
    out = _tc_fin(S2[0], S2[1], dis, b2.reshape(1, H))
    return out[:N]
